# Initial kernel scaffold; baseline (speedup 1.0000x reference)
import jax
import jax.numpy as jnp
from jax.experimental import pallas as pl


def kernel(points_bary, face_ids, faces_uvs_index):
    raise NotImplementedError("write your pallas kernel here")


# speedup vs baseline: 4.4760x; 4.4760x over previous
import functools

import jax
import jax.numpy as jnp
from jax import lax
from jax.experimental import pallas as pl
from jax.experimental.pallas import tpu as pltpu
from jax.experimental.pallas import tpu_sc as plsc

N_POINTS = 1048576
N_FACES = 1538

NUM_CORES = 2
NUM_SUBCORES = 16
NW = NUM_CORES * NUM_SUBCORES
PTS_PER_W = N_POINTS // NW
CHUNK = 8192
N_CHUNKS = PTS_PER_W // CHUNK
GROUPS = CHUNK // 16

_mesh = plsc.VectorSubcoreMesh(
    core_axis_name="c", subcore_axis_name="s", num_cores=NUM_CORES
)


@functools.partial(
    pl.kernel,
    out_type=jax.ShapeDtypeStruct((N_POINTS * 2,), jnp.float32),
    mesh=_mesh,
    compiler_params=pltpu.CompilerParams(needs_layout_passes=False),
    scratch_types=[
        pltpu.VMEM((N_FACES * 6,), jnp.float32),
        pltpu.VMEM((CHUNK,), jnp.int32),
        pltpu.VMEM((CHUNK * 3,), jnp.float32),
        pltpu.VMEM((CHUNK * 2,), jnp.float32),
    ],
)
def _uv_kernel(table_hbm, fid_hbm, bary_hbm, out_hbm, table_v, fid_v, bary_v, out_v):
    wid = lax.axis_index("s") * NUM_CORES + lax.axis_index("c")
    pltpu.sync_copy(table_hbm, table_v)

    lane = lax.iota(jnp.int32, 16)
    lane3 = lane * 3
    lane2 = lane * 2

    def chunk_body(ci, _):
        base = wid * PTS_PER_W + ci * CHUNK
        pltpu.sync_copy(fid_hbm.at[pl.ds(base, CHUNK)], fid_v)
        pltpu.sync_copy(bary_hbm.at[pl.ds(base * 3, CHUNK * 3)], bary_v)

        def group_body(g, _):
            gb = g * 16
            fid = fid_v[pl.ds(gb, 16)]
            tix = fid * 6
            t0 = plsc.load_gather(table_v, [tix])
            t1 = plsc.load_gather(table_v, [tix + 1])
            t2 = plsc.load_gather(table_v, [tix + 2])
            t3 = plsc.load_gather(table_v, [tix + 3])
            t4 = plsc.load_gather(table_v, [tix + 4])
            t5 = plsc.load_gather(table_v, [tix + 5])
            bix = gb * 3 + lane3
            b0 = plsc.load_gather(bary_v, [bix])
            b1 = plsc.load_gather(bary_v, [bix + 1])
            b2 = plsc.load_gather(bary_v, [bix + 2])
            u = t0 * b0 + t2 * b1 + t4 * b2
            v = t1 * b0 + t3 * b1 + t5 * b2
            oix = gb * 2 + lane2
            plsc.store_scatter(out_v, [oix], u)
            plsc.store_scatter(out_v, [oix + 1], v)
            return 0

        lax.fori_loop(0, GROUPS, group_body, 0, unroll=4)
        pltpu.sync_copy(out_v, out_hbm.at[pl.ds(base * 2, CHUNK * 2)])
        return 0

    lax.fori_loop(0, N_CHUNKS, chunk_body, 0)


def kernel(points_bary, face_ids, faces_uvs_index):
    table = faces_uvs_index.reshape(-1)
    fid = face_ids.astype(jnp.int32)
    bary = points_bary.reshape(-1)
    out = _uv_kernel(table, fid, bary)
    return out.reshape(N_POINTS, 2)
